# speedup vs baseline: 1.4490x; 1.4490x over previous
import jax
import jax.numpy as jnp
from jax.experimental import pallas as pl

_ROWS = 16384
_COLS = 128
_BLOCK_ROWS = 1024


def _gather_cols_kernel(x_ref, o_ref):
    x = x_ref[...]
    o_ref[...] = jnp.concatenate(
        [x[:, 0:1], x[:, 1:2], x[:, 4:5], x[:, 4:5]], axis=1
    )


def kernel(x):
    grid = (_ROWS // _BLOCK_ROWS,)
    return pl.pallas_call(
        _gather_cols_kernel,
        grid=grid,
        in_specs=[pl.BlockSpec((_BLOCK_ROWS, _COLS), lambda i: (i, 0))],
        out_specs=pl.BlockSpec((_BLOCK_ROWS, 4), lambda i: (i, 0)),
        out_shape=jax.ShapeDtypeStruct((_ROWS, 4), jnp.float32),
    )(x)
